# baseline (speedup 1.0000x reference)
import functools
import jax
import jax.numpy as jnp
from jax import lax
from jax.experimental import pallas as pl
from jax.experimental.pallas import tpu as pltpu
from jax.experimental.pallas import tpu_sc as plsc

B = 64
S = 576
R = S + 1
D = 768
NW = 32
RW = S // NW
LANES = 16
KSLICES = D // LANES


def _sc_body(in_hbm, ce_hbm, pe0_hbm, pem_hbm, out_hbm,
             pe_buf, io_buf, row0_buf, ce_buf):
    cid = lax.axis_index("c")
    sid = lax.axis_index("s")
    w = sid * 2 + cid

    pltpu.sync_copy(pem_hbm.at[pl.ds(w * RW, RW)], pe_buf)

    def batch_body(b, carry):
        in_base = b * S + w * RW
        out_base = b * R + 1 + w * RW
        pltpu.sync_copy(in_hbm.at[pl.ds(in_base, RW)], io_buf)

        def row_body(i, c2):
            for k in range(KSLICES):
                plsc.addupdate(io_buf.at[i, pl.ds(k * LANES, LANES)],
                               pe_buf[i, pl.ds(k * LANES, LANES)])
            return c2

        lax.fori_loop(0, RW, row_body, 0)
        pltpu.sync_copy(io_buf, out_hbm.at[pl.ds(out_base, RW)])
        return carry

    lax.fori_loop(0, B, batch_body, 0)

    @pl.when(w == 0)
    def _():
        pltpu.sync_copy(pe0_hbm, row0_buf)
        pltpu.sync_copy(ce_hbm, ce_buf)
        for k in range(KSLICES):
            plsc.addupdate(row0_buf.at[0, pl.ds(k * LANES, LANES)],
                           ce_buf[0, pl.ds(k * LANES, LANES)])

        def cls_body(b, carry):
            pltpu.sync_copy(row0_buf, out_hbm.at[pl.ds(b * R, 1)])
            return carry

        lax.fori_loop(0, B, cls_body, 0)


def kernel(inputs, class_embed, position_table):
    in_flat = inputs.reshape(B * S, D)
    ce = class_embed.reshape(1, D)
    pe0 = position_table[0:1]
    pem = position_table[1:R]

    mesh = plsc.VectorSubcoreMesh(core_axis_name="c", subcore_axis_name="s",
                                  num_cores=2)
    run = functools.partial(
        pl.kernel,
        mesh=mesh,
        out_type=jax.ShapeDtypeStruct((B * R, D), jnp.float32),
        scratch_types=[
            pltpu.VMEM((RW, D), jnp.float32),
            pltpu.VMEM((RW, D), jnp.float32),
            pltpu.VMEM((1, D), jnp.float32),
            pltpu.VMEM((1, D), jnp.float32),
        ],
        compiler_params=pltpu.CompilerParams(use_tc_tiling_on_sc=False),
    )(_sc_body)
    out_flat = run(in_flat, ce, pe0, pem)
    return out_flat.reshape(B, R, D)


# speedup vs baseline: 1.3907x; 1.3907x over previous
import functools
import jax
import jax.numpy as jnp
from jax import lax
from jax.experimental import pallas as pl
from jax.experimental.pallas import tpu as pltpu
from jax.experimental.pallas import tpu_sc as plsc

B = 64
S = 576
R = S + 1
D = 768
NW = 32
C = 24
NCHUNK = S // C
NB_W = B // NW
NSTEP = NB_W * NCHUNK
PE_ROWS = 576
LANES = 16
KS = D // LANES


def _row_add(dst_ref, dst_row, a_ref, a_row, b_ref, b_row):
    for j in range(KS):
        sl = pl.ds(j * LANES, LANES)
        dst_ref[dst_row, sl] = a_ref[a_row, sl] + b_ref[b_row, sl]


def _row_copy(dst_ref, dst_row, src_ref, src_row):
    for j in range(KS):
        sl = pl.ds(j * LANES, LANES)
        dst_ref[dst_row, sl] = src_ref[src_row, sl]


def _sc_body(in_hbm, ce_hbm, tab_hbm, out_hbm,
             io0, io1, pe0, pe1, ce_buf, save, stash, tpe_buf, tail_buf,
             pe_sp, s_in0, s_in1, s_pe0, s_pe1, s_out0, s_out1):
    cid = lax.axis_index("c")
    sid = lax.axis_index("s")
    w = sid * 2 + cid

    @pl.when(sid == 0)
    def _():
        pltpu.sync_copy(tab_hbm.at[pl.ds(0, PE_ROWS)], pe_sp)

    plsc.subcore_barrier()

    pltpu.sync_copy(ce_hbm.at[0], ce_buf)
    pltpu.sync_copy(tab_hbm.at[pl.ds(S, 8)], tpe_buf)

    def in_copy(t, io, sem):
        b = 2 * w + t // NCHUNK
        k = t % NCHUNK
        return pltpu.make_async_copy(
            in_hbm.at[b, pl.ds(k * C, C)], io, sem)

    def pe_copy(t, pe, sem):
        k = t % NCHUNK
        return pltpu.make_async_copy(pe_sp.at[pl.ds(k * C, C)], pe, sem)

    def out_copy(t, io, sem):
        b = 2 * w + t // NCHUNK
        k = t % NCHUNK
        return pltpu.make_async_copy(
            io, out_hbm.at[b, pl.ds(k * C, C)], sem)

    def step(t, io, pe, s_in, s_pe, s_out, io_n, pe_n, s_in_n, s_pe_n,
             s_out_n):
        b = 2 * w + t // NCHUNK
        k = t % NCHUNK

        @pl.when(t >= 1)
        def _():
            out_copy(t - 1, io_n, s_out_n).wait()

        @pl.when(t + 1 < NSTEP)
        def _():
            in_copy(t + 1, io_n, s_in_n).start()
            pe_copy(t + 1, pe_n, s_pe_n).start()

        in_copy(t, io, s_in).wait()
        pe_copy(t, pe, s_pe).wait()

        @pl.when(k == 0)
        def _():
            _row_copy(save, 0, ce_buf, 0)

        _row_copy(stash, 0, io, C - 1)

        @pl.when(k == NCHUNK - 1)
        def _():
            _row_add(tail_buf, 0, tpe_buf, 0, stash, 0)
            pltpu.sync_copy(tail_buf, out_hbm.at[b, pl.ds(S, 1)])

        def row_body(i, carry):
            row = (C - 1) - i
            _row_add(io, row, pe, row, io, row - 1)
            return carry

        lax.fori_loop(0, C - 1, row_body, 0)
        _row_add(io, 0, pe, 0, save, 0)
        _row_copy(save, 0, stash, 0)

        out_copy(t, io, s_out).start()

    in_copy(0, io0, s_in0).start()
    pe_copy(0, pe0, s_pe0).start()

    def pair_body(tp, carry):
        t0 = 2 * tp
        step(t0, io0, pe0, s_in0, s_pe0, s_out0,
             io1, pe1, s_in1, s_pe1, s_out1)
        step(t0 + 1, io1, pe1, s_in1, s_pe1, s_out1,
             io0, pe0, s_in0, s_pe0, s_out0)
        return carry

    lax.fori_loop(0, NSTEP // 2, pair_body, 0)
    out_copy(NSTEP - 1, io1, s_out1).wait()


def kernel(inputs, class_embed, position_table):
    mesh = plsc.VectorSubcoreMesh(core_axis_name="c", subcore_axis_name="s",
                                  num_cores=2)
    run = functools.partial(
        pl.kernel,
        mesh=mesh,
        out_type=jax.ShapeDtypeStruct((B, R, D), jnp.float32),
        scratch_types=[
            pltpu.VMEM((C, D), jnp.float32),
            pltpu.VMEM((C, D), jnp.float32),
            pltpu.VMEM((C, D), jnp.float32),
            pltpu.VMEM((C, D), jnp.float32),
            pltpu.VMEM((1, D), jnp.float32),
            pltpu.VMEM((1, D), jnp.float32),
            pltpu.VMEM((1, D), jnp.float32),
            pltpu.VMEM((8, D), jnp.float32),
            pltpu.VMEM((1, D), jnp.float32),
            pltpu.VMEM_SHARED((PE_ROWS, D), jnp.float32),
            pltpu.SemaphoreType.DMA,
            pltpu.SemaphoreType.DMA,
            pltpu.SemaphoreType.DMA,
            pltpu.SemaphoreType.DMA,
            pltpu.SemaphoreType.DMA,
            pltpu.SemaphoreType.DMA,
        ],
    )(_sc_body)
    return run(inputs, class_embed, position_table)


# speedup vs baseline: 1.7498x; 1.2582x over previous
import functools
import jax
import jax.numpy as jnp
from jax import lax
from jax.experimental import pallas as pl
from jax.experimental.pallas import tpu as pltpu
from jax.experimental.pallas import tpu_sc as plsc

B = 64
S = 576
R = S + 1
D = 768
NW = 32
C = 24
NCHUNK = S // C
NB_W = B // NW
NSTEP = NB_W * NCHUNK
PE_ROWS = 576
LANES = 16
KS = D // LANES


def _row_add(dst_ref, dst_row, a_ref, a_row, b_ref, b_row):
    for j in range(KS):
        sl = pl.ds(j * LANES, LANES)
        dst_ref[dst_row, sl] = a_ref[a_row, sl] + b_ref[b_row, sl]


def _row_copy(dst_ref, dst_row, src_ref, src_row):
    for j in range(KS):
        sl = pl.ds(j * LANES, LANES)
        dst_ref[dst_row, sl] = src_ref[src_row, sl]


def _row_addupdate(dst_ref, dst_row, src_ref, src_row):
    for j in range(KS):
        sl = pl.ds(j * LANES, LANES)
        plsc.addupdate(dst_ref.at[dst_row, sl], src_ref[src_row, sl])


def _sc_body(in_hbm, ce_hbm, tab_hbm, out_hbm,
             io0, io1, pe0, pe1, pe2, ce_buf, save, stash, tpe_buf, tail_buf,
             pe_sp, s_in0, s_in1, s_pe0, s_pe1, s_pe2, s_out0, s_out1,
             s_out2):
    cid = lax.axis_index("c")
    sid = lax.axis_index("s")
    w = sid * 2 + cid

    ios = (io0, io1)
    pes = (pe0, pe1, pe2)
    s_ins = (s_in0, s_in1)
    s_pes = (s_pe0, s_pe1, s_pe2)
    s_outs = (s_out0, s_out1, s_out2)

    @pl.when(sid == 0)
    def _():
        pltpu.sync_copy(tab_hbm.at[pl.ds(0, PE_ROWS)], pe_sp)

    plsc.subcore_barrier()

    pltpu.sync_copy(ce_hbm.at[0], ce_buf)
    pltpu.sync_copy(tab_hbm.at[pl.ds(S, 8)], tpe_buf)

    def in_copy(t, io, sem):
        b = 2 * w + t // NCHUNK
        k = t % NCHUNK
        return pltpu.make_async_copy(
            in_hbm.at[b, pl.ds(k * C, C)], io, sem)

    def pe_copy(t, pe, sem):
        k = t % NCHUNK
        return pltpu.make_async_copy(pe_sp.at[pl.ds(k * C, C)], pe, sem)

    def out_copy(t, pe, sem):
        b = 2 * w + t // NCHUNK
        k = t % NCHUNK
        return pltpu.make_async_copy(
            pe, out_hbm.at[b, pl.ds(k * C, C)], sem)

    def step(t, v):
        io, io_n = ios[v % 2], ios[(v + 1) % 2]
        pe, pe_n = pes[v % 3], pes[(v + 1) % 3]
        b = 2 * w + t // NCHUNK
        k = t % NCHUNK

        @pl.when(t >= 2)
        def _():
            out_copy(t - 2, pe_n, s_outs[(v + 1) % 3]).wait()

        @pl.when(t + 1 < NSTEP)
        def _():
            in_copy(t + 1, io_n, s_ins[(v + 1) % 2]).start()
            pe_copy(t + 1, pe_n, s_pes[(v + 1) % 3]).start()

        in_copy(t, io, s_ins[v % 2]).wait()
        pe_copy(t, pe, s_pes[v % 3]).wait()

        @pl.when(k == 0)
        def _():
            _row_copy(save, 0, ce_buf, 0)

        _row_copy(stash, 0, io, C - 1)

        @pl.when(k == NCHUNK - 1)
        def _():
            _row_add(tail_buf, 0, tpe_buf, 0, stash, 0)
            pltpu.sync_copy(tail_buf, out_hbm.at[b, pl.ds(S, 1)])

        def row_body(i, carry):
            _row_addupdate(pe, i, io, i - 1)
            return carry

        lax.fori_loop(1, C, row_body, 0)
        _row_addupdate(pe, 0, save, 0)
        _row_copy(save, 0, stash, 0)

        out_copy(t, pe, s_outs[v % 3]).start()

    in_copy(0, io0, s_in0).start()
    pe_copy(0, pe0, s_pe0).start()

    def six_body(u, carry):
        t0 = 6 * u
        for v in range(6):
            step(t0 + v, v)
        return carry

    lax.fori_loop(0, NSTEP // 6, six_body, 0)
    out_copy(NSTEP - 2, pes[(NSTEP - 2) % 3], s_outs[(NSTEP - 2) % 3]).wait()
    out_copy(NSTEP - 1, pes[(NSTEP - 1) % 3], s_outs[(NSTEP - 1) % 3]).wait()


def kernel(inputs, class_embed, position_table):
    mesh = plsc.VectorSubcoreMesh(core_axis_name="c", subcore_axis_name="s",
                                  num_cores=2)
    run = functools.partial(
        pl.kernel,
        mesh=mesh,
        out_type=jax.ShapeDtypeStruct((B, R, D), jnp.float32),
        scratch_types=[
            pltpu.VMEM((C, D), jnp.float32),
            pltpu.VMEM((C, D), jnp.float32),
            pltpu.VMEM((C, D), jnp.float32),
            pltpu.VMEM((C, D), jnp.float32),
            pltpu.VMEM((C, D), jnp.float32),
            pltpu.VMEM((1, D), jnp.float32),
            pltpu.VMEM((1, D), jnp.float32),
            pltpu.VMEM((1, D), jnp.float32),
            pltpu.VMEM((8, D), jnp.float32),
            pltpu.VMEM((1, D), jnp.float32),
            pltpu.VMEM_SHARED((PE_ROWS, D), jnp.float32),
            pltpu.SemaphoreType.DMA,
            pltpu.SemaphoreType.DMA,
            pltpu.SemaphoreType.DMA,
            pltpu.SemaphoreType.DMA,
            pltpu.SemaphoreType.DMA,
            pltpu.SemaphoreType.DMA,
            pltpu.SemaphoreType.DMA,
            pltpu.SemaphoreType.DMA,
        ],
    )(_sc_body)
    return run(inputs, class_embed, position_table)


# speedup vs baseline: 3.6000x; 2.0573x over previous
import jax
import jax.numpy as jnp
from jax.experimental import pallas as pl


_BB = 8


def _body(in_ref, ce_ref, pe0_ref, pe_ref, out_ref):
    row0 = ce_ref[0] + pe0_ref[...]
    for j in range(_BB):
        out_ref[j, 0:1, :] = row0
        out_ref[j, 1:, :] = in_ref[j] + pe_ref[...]


def kernel(inputs, class_embed, position_table):
    B, S, D = inputs.shape
    pe0 = position_table[0:1]
    pe = position_table[1:S + 1]
    return pl.pallas_call(
        _body,
        grid=(B // _BB,),
        in_specs=[
            pl.BlockSpec((_BB, S, D), lambda b: (b, 0, 0)),
            pl.BlockSpec((1, 1, D), lambda b: (0, 0, 0)),
            pl.BlockSpec((1, D), lambda b: (0, 0)),
            pl.BlockSpec((S, D), lambda b: (0, 0)),
        ],
        out_specs=pl.BlockSpec((_BB, S + 1, D), lambda b: (b, 0, 0)),
        out_shape=jax.ShapeDtypeStruct((B, S + 1, D), jnp.float32),
    )(inputs, class_embed, pe0, pe)
